# initial kernel scaffold (unmeasured)
import jax
import jax.numpy as jnp
from jax import lax
from jax.experimental import pallas as pl
from jax.experimental.pallas import tpu as pltpu


def kernel(
    x,
):
    def body(*refs):
        pass

    out_shape = jax.ShapeDtypeStruct(..., jnp.float32)
    return pl.pallas_call(body, out_shape=out_shape)(...)


# baseline (device time: 46868 ns/iter reference)
import jax
import jax.numpy as jnp
from jax import lax
from jax.experimental import pallas as pl
from jax.experimental.pallas import tpu as pltpu

N_DEV = 4
M = 1024
N_FULL = 2048
N_PER = N_FULL // N_DEV


def kernel(x):
    def body(x_ref, out_ref, comm_ref, send_sems, recv_sems):
        my = lax.axis_index("i")
        left = lax.rem(my + N_DEV - 1, N_DEV)
        right = lax.rem(my + 1, N_DEV)

        barrier_sem = pltpu.get_barrier_semaphore()
        for nbr in (left, right):
            pl.semaphore_signal(
                barrier_sem, inc=1,
                device_id=(nbr,), device_id_type=pl.DeviceIdType.MESH,
            )
        pl.semaphore_wait(barrier_sem, 2)

        c0 = lax.rem(my + N_DEV - 1, N_DEV)
        comm_ref[0, :, :] = x_ref[0, :, pl.ds(c0 * N_PER, N_PER)].astype(
            jnp.bfloat16
        )

        for s in range(N_DEV - 1):
            rdma = pltpu.make_async_remote_copy(
                src_ref=comm_ref.at[s],
                dst_ref=comm_ref.at[s + 1],
                send_sem=send_sems.at[s],
                recv_sem=recv_sems.at[s],
                device_id=(right,),
                device_id_type=pl.DeviceIdType.MESH,
            )
            rdma.start()
            rdma.wait()
            cr = lax.rem(my + 2 * N_DEV - s - 2, N_DEV)
            comm_ref[s + 1, :, :] += x_ref[
                0, :, pl.ds(cr * N_PER, N_PER)
            ].astype(jnp.bfloat16)

        out_ref[:, :] = comm_ref[N_DEV - 1, :, :]

    return pl.pallas_call(
        body,
        out_shape=jax.ShapeDtypeStruct((M, N_PER), jnp.bfloat16),
        in_specs=[pl.BlockSpec(memory_space=pltpu.VMEM)],
        out_specs=pl.BlockSpec(memory_space=pltpu.VMEM),
        scratch_shapes=[
            pltpu.VMEM((N_DEV, M, N_PER), jnp.bfloat16),
            pltpu.SemaphoreType.DMA((N_DEV - 1,)),
            pltpu.SemaphoreType.DMA((N_DEV - 1,)),
        ],
        compiler_params=pltpu.CompilerParams(collective_id=0),
    )(x)


# device time: 26735 ns/iter; 1.7531x vs baseline; 1.7531x over previous
import jax
import jax.numpy as jnp
from jax import lax
from jax.experimental import pallas as pl
from jax.experimental.pallas import tpu as pltpu

N_DEV = 4
M = 1024
N_FULL = 2048
N_PER = N_FULL // N_DEV
HALF = N_PER // 2
K = 2
SEG = HALF // K
N_HOPS = N_DEV - 1


def kernel(x):
    chains = []
    for j in range(K):
        chains.append((+1, j * SEG))
        chains.append((-1, HALF + j * SEG))
    n_chains = len(chains)

    def body(x_ref, out_ref, comm_ref, send_sems, recv_sems):
        my = lax.axis_index("i")
        left = lax.rem(my + N_DEV - 1, N_DEV)
        right = lax.rem(my + 1, N_DEV)

        def seed_chunk(d):
            return lax.rem(my + N_DEV - d, N_DEV)

        def recv_chunk(d, s):
            return lax.rem(my + 2 * N_DEV - d * (s + 2), N_DEV)

        def rdma(c, s):
            d, _ = chains[c]
            return pltpu.make_async_remote_copy(
                src_ref=comm_ref.at[c, s],
                dst_ref=comm_ref.at[c, s + 1],
                send_sem=send_sems.at[c, s],
                recv_sem=recv_sems.at[c, s],
                device_id=(right if d == 1 else left,),
                device_id_type=pl.DeviceIdType.MESH,
            )

        def x_seg(chunk, off):
            return x_ref[0, :, pl.ds(chunk * N_PER + off, SEG)].astype(
                jnp.bfloat16
            )

        barrier_sem = pltpu.get_barrier_semaphore()
        for nbr in (left, right):
            pl.semaphore_signal(
                barrier_sem, inc=1,
                device_id=(nbr,), device_id_type=pl.DeviceIdType.MESH,
            )
        pl.semaphore_wait(barrier_sem, 2)

        for c, (d, off) in enumerate(chains):
            comm_ref[c, 0, :, :] = x_seg(seed_chunk(d), off)
            rdma(c, 0).start()

        for s in range(N_HOPS):
            for c, (d, off) in enumerate(chains):
                rdma(c, s).wait_recv()
                comm_ref[c, s + 1, :, :] += x_seg(recv_chunk(d, s), off)
                if s + 1 < N_HOPS:
                    rdma(c, s + 1).start()

        for c, (d, off) in enumerate(chains):
            out_ref[:, pl.ds(off, SEG)] = comm_ref[c, N_HOPS, :, :]

        for s in range(N_HOPS):
            for c in range(n_chains):
                rdma(c, s).wait_send()

    return pl.pallas_call(
        body,
        out_shape=jax.ShapeDtypeStruct((M, N_PER), jnp.bfloat16),
        in_specs=[pl.BlockSpec(memory_space=pltpu.VMEM)],
        out_specs=pl.BlockSpec(memory_space=pltpu.VMEM),
        scratch_shapes=[
            pltpu.VMEM((n_chains, N_DEV, M, SEG), jnp.bfloat16),
            pltpu.SemaphoreType.DMA((n_chains, N_HOPS)),
            pltpu.SemaphoreType.DMA((n_chains, N_HOPS)),
        ],
        compiler_params=pltpu.CompilerParams(collective_id=0),
    )(x)


# device time: 26577 ns/iter; 1.7635x vs baseline; 1.0059x over previous
import jax
import jax.numpy as jnp
from jax import lax
from jax.experimental import pallas as pl
from jax.experimental.pallas import tpu as pltpu

N_DEV = 4
M = 1024
N_FULL = 2048
N_PER = N_FULL // N_DEV
HALF = N_PER // 2
ROW_K = 4
SEG_M = M // ROW_K
N_HOPS = N_DEV - 1


def kernel(x):
    chains = []
    for j in range(ROW_K):
        chains.append((+1, 0, j * SEG_M))
        chains.append((-1, HALF, j * SEG_M))
    n_chains = len(chains)

    def body(x_ref, out_ref, comm_ref, send_sems, recv_sems):
        my = lax.axis_index("i")
        left = lax.rem(my + N_DEV - 1, N_DEV)
        right = lax.rem(my + 1, N_DEV)

        def seed_chunk(d):
            return lax.rem(my + N_DEV - d, N_DEV)

        def recv_chunk(d, s):
            return lax.rem(my + 2 * N_DEV - d * (s + 2), N_DEV)

        def rdma(c, s):
            d = chains[c][0]
            return pltpu.make_async_remote_copy(
                src_ref=comm_ref.at[c, s],
                dst_ref=comm_ref.at[c, s + 1],
                send_sem=send_sems.at[c, s],
                recv_sem=recv_sems.at[c, s],
                device_id=(right if d == 1 else left,),
                device_id_type=pl.DeviceIdType.MESH,
            )

        def x_seg(chunk, col_off, row_off):
            return x_ref[
                0, pl.ds(row_off, SEG_M), pl.ds(chunk * N_PER + col_off, HALF)
            ].astype(jnp.bfloat16)

        barrier_sem = pltpu.get_barrier_semaphore()
        for nbr in (left, right):
            pl.semaphore_signal(
                barrier_sem, inc=1,
                device_id=(nbr,), device_id_type=pl.DeviceIdType.MESH,
            )
        pl.semaphore_wait(barrier_sem, 2)

        for c, (d, co, ro) in enumerate(chains):
            comm_ref[c, 0, :, :] = x_seg(seed_chunk(d), co, ro)
            rdma(c, 0).start()

        for s in range(N_HOPS):
            for c, (d, co, ro) in enumerate(chains):
                rdma(c, s).wait_recv()
                comm_ref[c, s + 1, :, :] += x_seg(recv_chunk(d, s), co, ro)
                if s + 1 < N_HOPS:
                    rdma(c, s + 1).start()
                else:
                    out_ref[pl.ds(ro, SEG_M), pl.ds(co, HALF)] = comm_ref[
                        c, N_HOPS, :, :
                    ]

        for s in range(N_HOPS):
            for c in range(n_chains):
                rdma(c, s).wait_send()

    return pl.pallas_call(
        body,
        out_shape=jax.ShapeDtypeStruct((M, N_PER), jnp.bfloat16),
        in_specs=[pl.BlockSpec(memory_space=pltpu.VMEM)],
        out_specs=pl.BlockSpec(memory_space=pltpu.VMEM),
        scratch_shapes=[
            pltpu.VMEM((n_chains, N_DEV, SEG_M, HALF), jnp.bfloat16),
            pltpu.SemaphoreType.DMA((n_chains, N_HOPS)),
            pltpu.SemaphoreType.DMA((n_chains, N_HOPS)),
        ],
        compiler_params=pltpu.CompilerParams(collective_id=0),
    )(x)
